# baseline (speedup 1.0000x reference)
import functools

import jax
import jax.numpy as jnp
from jax import lax
from jax.experimental import pallas as pl
from jax.experimental.pallas import tpu as pltpu
from jax.experimental.pallas import tpu_sc as plsc

BATCH = 4096
SEQ = 50
EMB = 100

NC = 2
NS = 16
NW = NC * NS

TOTAL = BATCH * SEQ
B_PER_W = TOTAL // NW
CHUNK = 128
N_CHUNKS = B_PER_W // CHUNK


@functools.cache
def _build_gather_kernel():
    mesh = plsc.VectorSubcoreMesh(core_axis_name="c", subcore_axis_name="s")

    @functools.partial(
        pl.kernel,
        out_type=jax.ShapeDtypeStruct((TOTAL, EMB), jnp.float32),
        mesh=mesh,
        scratch_types=[
            pltpu.VMEM((N_CHUNKS, CHUNK), jnp.int32),
            pltpu.VMEM((CHUNK, EMB), jnp.float32),
            pltpu.VMEM((CHUNK, EMB), jnp.float32),
            pltpu.SemaphoreType.DMA,
            pltpu.SemaphoreType.DMA,
        ],
        compiler_params=pltpu.CompilerParams(use_tc_tiling_on_sc=False),
    )
    def _gather_kernel(idx_hbm, table_hbm, out_hbm, idx_v, rows0, rows1,
                       sem0, sem1):
        wid = lax.axis_index("s") * NC + lax.axis_index("c")
        base = wid * B_PER_W

        pltpu.sync_copy(idx_hbm.at[wid], idx_v)

        del rows1, sem1

        def loop_body(j, carry):
            del carry
            pltpu.async_copy(table_hbm.at[idx_v.at[j]], rows0, sem0).wait()
            pltpu.sync_copy(
                rows0, out_hbm.at[pl.ds(base + j * CHUNK, CHUNK)]
            )
            return 0

        lax.fori_loop(0, N_CHUNKS, loop_body, 0, unroll=False)

    return _gather_kernel


def kernel(indices, table):
    idx = indices.reshape(NW, N_CHUNKS, CHUNK).astype(jnp.int32)
    out = _build_gather_kernel()(idx, table)
    return out.reshape(BATCH, SEQ, EMB)


# speedup vs baseline: 2.1861x; 2.1861x over previous
import functools

import jax
import jax.numpy as jnp
from jax import lax
from jax.experimental import pallas as pl
from jax.experimental.pallas import tpu as pltpu
from jax.experimental.pallas import tpu_sc as plsc

BATCH = 4096
SEQ = 50
EMB = 100

NC = 2
NS = 16
NW = NC * NS

BATCHES_PER_TILE = BATCH // NW
TOKENS_PER_TILE = BATCHES_PER_TILE * SEQ
IDX_PAD = TOKENS_PER_TILE + 16


@functools.cache
def _build_gather_kernel():
    mesh = plsc.VectorSubcoreMesh(core_axis_name="c", subcore_axis_name="s")

    @functools.partial(
        pl.kernel,
        out_type=jax.ShapeDtypeStruct((BATCH, SEQ, EMB), jnp.float32),
        mesh=mesh,
        scratch_types=[
            pltpu.VMEM((IDX_PAD,), jnp.int32),
            pltpu.VMEM((SEQ, EMB), jnp.float32),
            pltpu.VMEM((SEQ, EMB), jnp.float32),
            pltpu.SemaphoreType.DMA,
            pltpu.SemaphoreType.DMA,
            pltpu.SemaphoreType.DMA,
            pltpu.SemaphoreType.DMA,
        ],
    )
    def k(idx_hbm, table_hbm, out_hbm, idx_v, buf0, buf1, g0, g1, w0, w1):
        wid = lax.axis_index("s") * NC + lax.axis_index("c")
        b_first = wid * BATCHES_PER_TILE

        pltpu.sync_copy(idx_hbm.at[wid], idx_v.at[pl.ds(0, TOKENS_PER_TILE)])

        bufs = (buf0, buf1)
        gsems = (g0, g1)
        wsems = (w0, w1)

        def batch_body(b, carry):
            del carry
            for par in range(2):
                @pl.when(lax.rem(b, 2) == par)
                def _():
                    buf, gsem, wsem = bufs[par], gsems[par], wsems[par]
                    @pl.when(b >= 2)
                    def _():
                        pltpu.make_async_copy(buf, out_hbm.at[0], wsem).wait()
                    base = b * SEQ
                    vecs = [idx_v[pl.ds(base + off, 16)]
                            for off in (0, 16, 32, 48)]
                    for t in range(SEQ):
                        v, lane = divmod(t, 16)
                        row = vecs[v][lane]
                        pltpu.async_copy(
                            table_hbm.at[pl.ds(row, 1)],
                            buf.at[pl.ds(t, 1)], gsem)
                    pltpu.make_async_copy(out_hbm.at[0], buf, gsem).wait()
                    pltpu.async_copy(buf, out_hbm.at[b_first + b], wsem)
            return 0

        lax.fori_loop(0, BATCHES_PER_TILE, batch_body, 0, unroll=False)
        pltpu.make_async_copy(buf0, out_hbm.at[0], w0).wait()
        pltpu.make_async_copy(buf1, out_hbm.at[0], w1).wait()

    return k


def kernel(indices, table):
    idx = indices.astype(jnp.int32).reshape(NW, TOKENS_PER_TILE)
    return _build_gather_kernel()(idx, table)


# speedup vs baseline: 2.5842x; 1.1821x over previous
import functools

import jax
import jax.numpy as jnp
from jax import lax
from jax.experimental import pallas as pl
from jax.experimental.pallas import tpu as pltpu
from jax.experimental.pallas import tpu_sc as plsc

BATCH = 4096
SEQ = 50
EMB = 100

NC = 2
NS = 16
NW = NC * NS

BATCHES_PER_TILE = BATCH // NW
TOKENS_PER_TILE = BATCHES_PER_TILE * SEQ
IDX_PAD = TOKENS_PER_TILE + 16
NBUF = 4


@functools.cache
def _build_gather_kernel():
    mesh = plsc.VectorSubcoreMesh(core_axis_name="c", subcore_axis_name="s")

    @functools.partial(
        pl.kernel,
        out_type=jax.ShapeDtypeStruct((BATCH, SEQ, EMB), jnp.float32),
        mesh=mesh,
        scratch_types=[
            pltpu.VMEM((IDX_PAD,), jnp.int32),
            pltpu.VMEM((NBUF, SEQ, EMB), jnp.float32),
            [pltpu.SemaphoreType.DMA] * NBUF,
            [pltpu.SemaphoreType.DMA] * NBUF,
        ],
    )
    def k(idx_hbm, table_hbm, out_hbm, idx_v, bufs, gsems, wsems):
        wid = lax.axis_index("s") * NC + lax.axis_index("c")
        b_first = wid * BATCHES_PER_TILE

        pltpu.sync_copy(idx_hbm.at[wid], idx_v.at[pl.ds(0, TOKENS_PER_TILE)])

        def issue_gathers(b, j):
            base = b * SEQ
            vecs = [idx_v[pl.ds(base + off, 16)] for off in (0, 16, 32, 48)]
            for t in range(SEQ):
                v, lane = divmod(t, 16)
                pltpu.async_copy(
                    table_hbm.at[pl.ds(vecs[v][lane], 1)],
                    bufs.at[j, pl.ds(t, 1)], gsems[j])

        for j in range(NBUF):
            issue_gathers(j, j)

        def body(i, carry):
            del carry
            for j in range(NBUF):
                b = NBUF * i + j
                pltpu.make_async_copy(out_hbm.at[0], bufs.at[0], gsems[j]).wait()
                pltpu.async_copy(bufs.at[j], out_hbm.at[b_first + b], wsems[j])
            for j in range(NBUF):
                @pl.when(i < BATCHES_PER_TILE // NBUF - 1)
                def _():
                    pltpu.make_async_copy(
                        bufs.at[0], out_hbm.at[0], wsems[j]).wait()
                    issue_gathers(NBUF * (i + 1) + j, j)
            return 0

        lax.fori_loop(0, BATCHES_PER_TILE // NBUF, body, 0, unroll=False)
        for j in range(NBUF):
            pltpu.make_async_copy(bufs.at[0], out_hbm.at[0], wsems[j]).wait()

    return k


def kernel(indices, table):
    idx = indices.astype(jnp.int32).reshape(NW, TOKENS_PER_TILE)
    return _build_gather_kernel()(idx, table)


# speedup vs baseline: 2.5921x; 1.0031x over previous
import functools

import jax
import jax.numpy as jnp
from jax import lax
from jax.experimental import pallas as pl
from jax.experimental.pallas import tpu as pltpu
from jax.experimental.pallas import tpu_sc as plsc

BATCH = 4096
SEQ = 50
EMB = 100

NC = 2
NS = 16
NW = NC * NS

BATCHES_PER_TILE = BATCH // NW
TOKENS_PER_TILE = BATCHES_PER_TILE * SEQ
IDX_PAD = TOKENS_PER_TILE + 16
NBUF = 8


@functools.cache
def _build_gather_kernel():
    mesh = plsc.VectorSubcoreMesh(core_axis_name="c", subcore_axis_name="s")

    @functools.partial(
        pl.kernel,
        out_type=jax.ShapeDtypeStruct((BATCH, SEQ, EMB), jnp.float32),
        mesh=mesh,
        scratch_types=[
            pltpu.VMEM((IDX_PAD,), jnp.int32),
            pltpu.VMEM((NBUF, SEQ, EMB), jnp.float32),
            [pltpu.SemaphoreType.DMA] * NBUF,
            [pltpu.SemaphoreType.DMA] * NBUF,
        ],
    )
    def k(idx_hbm, table_hbm, out_hbm, idx_v, bufs, gsems, wsems):
        wid = lax.axis_index("s") * NC + lax.axis_index("c")
        b_first = wid * BATCHES_PER_TILE

        pltpu.sync_copy(idx_hbm.at[wid], idx_v.at[pl.ds(0, TOKENS_PER_TILE)])

        def issue_gathers(b, j):
            base = b * SEQ
            vecs = [idx_v[pl.ds(base + off, 16)] for off in (0, 16, 32, 48)]
            for t in range(SEQ):
                v, lane = divmod(t, 16)
                pltpu.async_copy(
                    table_hbm.at[pl.ds(vecs[v][lane], 1)],
                    bufs.at[j, pl.ds(t, 1)], gsems[j])

        for j in range(NBUF):
            issue_gathers(j, j)

        def body(i, carry):
            del carry
            for j in range(NBUF):
                b = NBUF * i + j
                pltpu.make_async_copy(out_hbm.at[0], bufs.at[0], gsems[j]).wait()
                pltpu.async_copy(bufs.at[j], out_hbm.at[b_first + b], wsems[j])
            for j in range(NBUF):
                @pl.when(i < BATCHES_PER_TILE // NBUF - 1)
                def _():
                    pltpu.make_async_copy(
                        bufs.at[0], out_hbm.at[0], wsems[j]).wait()
                    issue_gathers(NBUF * (i + 1) + j, j)
            return 0

        lax.fori_loop(0, BATCHES_PER_TILE // NBUF, body, 0, unroll=False)
        for j in range(NBUF):
            pltpu.make_async_copy(bufs.at[0], out_hbm.at[0], wsems[j]).wait()

    return k


def kernel(indices, table):
    idx = indices.astype(jnp.int32).reshape(NW, TOKENS_PER_TILE)
    return _build_gather_kernel()(idx, table)
